# Initial kernel scaffold; baseline (speedup 1.0000x reference)
import jax
import jax.numpy as jnp
from jax.experimental import pallas as pl


def kernel(node, edge_index, edge_attr, batch_ptr, eps, W1, b1, g1, be1, W2, b2, g2, be2, W3, b3, nw, nb):
    raise NotImplementedError("write your pallas kernel here")


# speedup vs baseline: 9.4242x; 9.4242x over previous
import functools

import jax
import jax.numpy as jnp
from jax import lax
from jax.experimental import pallas as pl
from jax.experimental.pallas import tpu as pltpu
from jax.experimental.pallas import tpu_sc as plsc

_N = 10000
_E = 320000
_D = 128
_G = 16

_NC = 2
_NS = 16
_NW = _NC * _NS
_EPW = _E // _NW
_CHUNK = 80
_NCHUNK = _EPW // _CHUNK
_RPS = 624
_TAIL0 = _RPS * _NS
_TAILN = _N - _TAIL0

_B = 2000
_NB = _N // _B


def _sc_agg_body(node_hbm, edge_hbm, out_hbm,
                 ix0, ix1, buf0, buf1, agg_sh, semi0, semi1, semg0, semg1):
    c = lax.axis_index("c")
    s = lax.axis_index("s")
    wid = c * _NS + s
    r0 = s * _RPS
    pltpu.sync_copy(node_hbm.at[pl.ds(r0, _RPS)], agg_sh.at[pl.ds(r0, _RPS)])

    @pl.when(s == 0)
    def _():
        pltpu.sync_copy(node_hbm.at[pl.ds(_TAIL0, _TAILN)],
                        agg_sh.at[pl.ds(_TAIL0, _TAILN)])

    plsc.subcore_barrier()

    pltpu.async_copy(edge_hbm.at[wid, 0], ix0, semi0)
    pltpu.async_copy(edge_hbm.at[wid, 1], ix1, semi1)
    pltpu.make_async_copy(edge_hbm.at[wid, 0], ix0, semi0).wait()
    pltpu.async_copy(node_hbm.at[ix0.at[0]], buf0, semg0)

    def pair(i, carry):
        j = 2 * i
        pltpu.make_async_copy(edge_hbm.at[wid, 0], ix1, semi1).wait()
        pltpu.async_copy(node_hbm.at[ix1.at[0]], buf1, semg1)
        pltpu.make_async_copy(node_hbm.at[ix0.at[0]], buf0, semg0).wait()
        pltpu.sync_copy(buf0, agg_sh.at[ix0.at[1]], add=True)

        @pl.when(j + 2 < _NCHUNK)
        def _():
            pltpu.async_copy(edge_hbm.at[wid, j + 2], ix0, semi0)
            pltpu.make_async_copy(edge_hbm.at[wid, 0], ix0, semi0).wait()
            pltpu.async_copy(node_hbm.at[ix0.at[0]], buf0, semg0)

        pltpu.make_async_copy(node_hbm.at[ix1.at[0]], buf1, semg1).wait()
        pltpu.sync_copy(buf1, agg_sh.at[ix1.at[1]], add=True)

        @pl.when(j + 3 < _NCHUNK)
        def _():
            pltpu.async_copy(edge_hbm.at[wid, j + 3], ix1, semi1)

        return carry

    lax.fori_loop(0, _NCHUNK // 2, pair, 0)
    pltpu.make_async_copy(node_hbm.at[ix0.at[0]], buf0, semg0).wait()
    pltpu.sync_copy(buf0, agg_sh.at[ix0.at[1]], add=True)

    plsc.subcore_barrier()
    pltpu.sync_copy(agg_sh.at[pl.ds(r0, _RPS)], out_hbm.at[c, pl.ds(r0, _RPS)])

    @pl.when(s == 0)
    def _():
        pltpu.sync_copy(agg_sh.at[pl.ds(_TAIL0, _TAILN)],
                        out_hbm.at[c, pl.ds(_TAIL0, _TAILN)])


@functools.cache
def _sc_agg():
    return pl.kernel(
        _sc_agg_body,
        out_type=jax.ShapeDtypeStruct((_NC, _N, _D), jnp.float32),
        mesh=plsc.VectorSubcoreMesh(core_axis_name="c", subcore_axis_name="s"),
        scratch_types=[
            pltpu.VMEM((2, _CHUNK), jnp.int32),
            pltpu.VMEM((2, _CHUNK), jnp.int32),
            pltpu.VMEM((_CHUNK, _D), jnp.float32),
            pltpu.VMEM((_CHUNK, _D), jnp.float32),
            pltpu.VMEM_SHARED((_N, _D), jnp.float32),
            pltpu.SemaphoreType.DMA,
            pltpu.SemaphoreType.DMA,
            pltpu.SemaphoreType.DMA,
            pltpu.SemaphoreType.DMA,
        ],
    )


def _ln(x, g, b):
    m = jnp.mean(x, axis=-1, keepdims=True)
    xc = x - m
    v = jnp.mean(xc * xc, axis=-1, keepdims=True)
    return xc * lax.rsqrt(v + 1e-5) * g + b


def _tc1_body(node_ref, p_ref, bp_ref, em1_ref,
              w1_ref, b1_ref, g1_ref, be1_ref,
              w2_ref, b2_ref, g2_ref, be2_ref,
              w3_ref, b3_ref, h3_ref, stats_ref):
    i = pl.program_id(0)
    h = em1_ref[0, 0] * node_ref[...] + p_ref[0] + p_ref[1]
    h = jnp.maximum(_ln(jnp.dot(h, w1_ref[...],
                                preferred_element_type=jnp.float32)
                        + b1_ref[...], g1_ref[...], be1_ref[...]), 0.0)
    h = jnp.maximum(_ln(jnp.dot(h, w2_ref[...],
                                preferred_element_type=jnp.float32)
                        + b2_ref[...], g2_ref[...], be2_ref[...]), 0.0)
    h = jnp.dot(h, w3_ref[...], preferred_element_type=jnp.float32) \
        + b3_ref[...]
    h3_ref[...] = h
    onehot = (bp_ref[...] == lax.broadcasted_iota(
        jnp.int32, (_B, _G), 1)).astype(jnp.float32)
    rs = jnp.sum(h, axis=1, keepdims=True)
    rs2 = jnp.sum(h * h, axis=1, keepdims=True)
    sum_g = jnp.sum(onehot * rs, axis=0)
    sum2_g = jnp.sum(onehot * rs2, axis=0)
    cnt_g = jnp.sum(onehot, axis=0)
    part = jnp.concatenate(
        [sum_g[:, None], sum2_g[:, None], cnt_g[:, None],
         jnp.zeros((_G, _D - 3), jnp.float32)], axis=1)

    @pl.when(i == 0)
    def _():
        stats_ref[...] = part

    @pl.when(i > 0)
    def _():
        stats_ref[...] += part


def _tc2_body(h3_ref, bp_ref, stats_ref, nw_ref, nb_ref, out_ref):
    stats = stats_ref[...]
    cnt = jnp.maximum(stats[:, 2] * jnp.float32(_D), 1.0)
    mean = stats[:, 0] / cnt
    var = jnp.maximum(stats[:, 1] / cnt - mean * mean, 0.0)
    inv = lax.rsqrt(var + 1e-5)
    onehot = (bp_ref[...] == lax.broadcasted_iota(
        jnp.int32, (_B, _G), 1)).astype(jnp.float32)
    mean_n = jnp.sum(onehot * mean[None, :], axis=1, keepdims=True)
    inv_n = jnp.sum(onehot * inv[None, :], axis=1, keepdims=True)
    out = (h3_ref[...] - mean_n) * inv_n * nw_ref[...] + nb_ref[...]
    out_ref[...] = jnp.maximum(out, 0.0)


def _row_spec(i_map=lambda i: (i, 0)):
    return pl.BlockSpec((_B, _D), i_map)


def _full(shape):
    return pl.BlockSpec(shape, lambda i: tuple(0 for _ in shape))


def kernel(node, edge_index, edge_attr, batch_ptr, eps,
           W1, b1, g1, be1, W2, b2, g2, be2, W3, b3, nw, nb):
    del edge_attr
    edges = edge_index.reshape(2, _NW, _NCHUNK, _CHUNK).transpose(1, 2, 0, 3)
    partials = _sc_agg()(node, edges)

    bp2 = batch_ptr.reshape(_N, 1)
    em1 = (eps - 1.0).reshape(1, 1)

    h3, stats = pl.pallas_call(
        _tc1_body,
        grid=(_NB,),
        in_specs=[
            _row_spec(),
            pl.BlockSpec((_NC, _B, _D), lambda i: (0, i, 0)),
            pl.BlockSpec((_B, 1), lambda i: (i, 0)),
            pl.BlockSpec(memory_space=pltpu.SMEM),
            _full((_D, _D)), _full((1, _D)), _full((1, _D)), _full((1, _D)),
            _full((_D, _D)), _full((1, _D)), _full((1, _D)), _full((1, _D)),
            _full((_D, _D)), _full((1, _D)),
        ],
        out_specs=[
            _row_spec(),
            _full((_G, _D)),
        ],
        out_shape=[
            jax.ShapeDtypeStruct((_N, _D), jnp.float32),
            jax.ShapeDtypeStruct((_G, _D), jnp.float32),
        ],
    )(node, partials, bp2, em1,
      W1, b1.reshape(1, _D), g1.reshape(1, _D), be1.reshape(1, _D),
      W2, b2.reshape(1, _D), g2.reshape(1, _D), be2.reshape(1, _D),
      W3, b3.reshape(1, _D))

    out = pl.pallas_call(
        _tc2_body,
        grid=(_NB,),
        in_specs=[
            _row_spec(),
            pl.BlockSpec((_B, 1), lambda i: (i, 0)),
            _full((_G, _D)),
            _full((1, _D)), _full((1, _D)),
        ],
        out_specs=_row_spec(),
        out_shape=jax.ShapeDtypeStruct((_N, _D), jnp.float32),
    )(h3, bp2, stats, nw.reshape(1, _D), nb.reshape(1, _D))
    return out


# speedup vs baseline: 11.1408x; 1.1821x over previous
import functools

import jax
import jax.numpy as jnp
from jax import lax
from jax.experimental import pallas as pl
from jax.experimental.pallas import tpu as pltpu
from jax.experimental.pallas import tpu_sc as plsc

_N = 10000
_E = 320000
_D = 128
_G = 16

_NC = 2
_NS = 16
_NW = _NC * _NS
_EPW = _E // _NW
_CHUNK = 80
_NCHUNK = _EPW // _CHUNK
_RPS = 624
_TAIL0 = _RPS * _NS
_TAILN = _N - _TAIL0

_B = 2000
_NB = _N // _B


def _sc_agg_body(node_hbm, edge_hbm, out_hbm,
                 ix0, ix1, ix2, ix3, buf0, buf1, buf2, buf3, agg_sh,
                 si0, si1, si2, si3, sg0, sg1, sg2, sg3):
    c = lax.axis_index("c")
    s = lax.axis_index("s")
    wid = c * _NS + s
    r0 = s * _RPS
    pltpu.sync_copy(node_hbm.at[pl.ds(r0, _RPS)], agg_sh.at[pl.ds(r0, _RPS)])

    @pl.when(s == 0)
    def _():
        pltpu.sync_copy(node_hbm.at[pl.ds(_TAIL0, _TAILN)],
                        agg_sh.at[pl.ds(_TAIL0, _TAILN)])

    plsc.subcore_barrier()

    ixs = (ix0, ix1, ix2, ix3)
    bufs = (buf0, buf1, buf2, buf3)
    sis = (si0, si1, si2, si3)
    sgs = (sg0, sg1, sg2, sg3)

    def fire_idx(m, k):
        pltpu.async_copy(edge_hbm.at[wid, m], ixs[k], sis[k])

    def wait_idx(k):
        pltpu.make_async_copy(edge_hbm.at[wid, 0], ixs[k], sis[k]).wait()

    def fire_gather(k):
        pltpu.async_copy(node_hbm.at[ixs[k].at[0]], bufs[k], sgs[k])

    def wait_gather(k):
        pltpu.make_async_copy(node_hbm.at[ixs[k].at[0]], bufs[k], sgs[k]).wait()

    for k in range(4):
        fire_idx(k, k)
    for k in range(2):
        wait_idx(k)
        fire_gather(k)

    def quad(i, carry):
        j = 4 * i
        for k in range(4):
            m = j + k
            wait_gather(k)
            pltpu.sync_copy(bufs[k], agg_sh.at[ixs[k].at[1]], add=True)

            @pl.when(m + 4 < _NCHUNK)
            def _():
                fire_idx(m + 4, k)

            @pl.when(m + 2 < _NCHUNK)
            def _():
                wait_idx((k + 2) % 4)
                fire_gather((k + 2) % 4)

        return carry

    lax.fori_loop(0, (_NCHUNK - 1) // 4, quad, 0)
    wait_gather(0)
    pltpu.sync_copy(bufs[0], agg_sh.at[ixs[0].at[1]], add=True)

    plsc.subcore_barrier()
    pltpu.sync_copy(agg_sh.at[pl.ds(r0, _RPS)], out_hbm.at[c, pl.ds(r0, _RPS)])

    @pl.when(s == 0)
    def _():
        pltpu.sync_copy(agg_sh.at[pl.ds(_TAIL0, _TAILN)],
                        out_hbm.at[c, pl.ds(_TAIL0, _TAILN)])


@functools.cache
def _sc_agg():
    return pl.kernel(
        _sc_agg_body,
        out_type=jax.ShapeDtypeStruct((_NC, _N, _D), jnp.float32),
        mesh=plsc.VectorSubcoreMesh(core_axis_name="c", subcore_axis_name="s"),
        scratch_types=(
            [pltpu.VMEM((2, _CHUNK), jnp.int32)] * 4
            + [pltpu.VMEM((_CHUNK, _D), jnp.float32)] * 4
            + [pltpu.VMEM_SHARED((_N, _D), jnp.float32)]
            + [pltpu.SemaphoreType.DMA] * 8
        ),
    )


def _ln(x, g, b):
    m = jnp.mean(x, axis=-1, keepdims=True)
    xc = x - m
    v = jnp.mean(xc * xc, axis=-1, keepdims=True)
    return xc * lax.rsqrt(v + 1e-5) * g + b


def _tc1_body(node_ref, p_ref, bp_ref, em1_ref,
              w1_ref, b1_ref, g1_ref, be1_ref,
              w2_ref, b2_ref, g2_ref, be2_ref,
              w3_ref, b3_ref, h3_ref, stats_ref):
    i = pl.program_id(0)
    h = em1_ref[0, 0] * node_ref[...] + p_ref[0] + p_ref[1]
    h = jnp.maximum(_ln(jnp.dot(h, w1_ref[...],
                                preferred_element_type=jnp.float32)
                        + b1_ref[...], g1_ref[...], be1_ref[...]), 0.0)
    h = jnp.maximum(_ln(jnp.dot(h, w2_ref[...],
                                preferred_element_type=jnp.float32)
                        + b2_ref[...], g2_ref[...], be2_ref[...]), 0.0)
    h = jnp.dot(h, w3_ref[...], preferred_element_type=jnp.float32) \
        + b3_ref[...]
    h3_ref[...] = h
    onehot = (bp_ref[...] == lax.broadcasted_iota(
        jnp.int32, (_B, _G), 1)).astype(jnp.float32)
    rs = jnp.sum(h, axis=1, keepdims=True)
    rs2 = jnp.sum(h * h, axis=1, keepdims=True)
    sum_g = jnp.sum(onehot * rs, axis=0)
    sum2_g = jnp.sum(onehot * rs2, axis=0)
    cnt_g = jnp.sum(onehot, axis=0)
    part = jnp.concatenate(
        [sum_g[:, None], sum2_g[:, None], cnt_g[:, None],
         jnp.zeros((_G, _D - 3), jnp.float32)], axis=1)

    @pl.when(i == 0)
    def _():
        stats_ref[...] = part

    @pl.when(i > 0)
    def _():
        stats_ref[...] += part


def _tc2_body(h3_ref, bp_ref, stats_ref, nw_ref, nb_ref, out_ref):
    stats = stats_ref[...]
    cnt = jnp.maximum(stats[:, 2] * jnp.float32(_D), 1.0)
    mean = stats[:, 0] / cnt
    var = jnp.maximum(stats[:, 1] / cnt - mean * mean, 0.0)
    inv = lax.rsqrt(var + 1e-5)
    onehot = (bp_ref[...] == lax.broadcasted_iota(
        jnp.int32, (_B, _G), 1)).astype(jnp.float32)
    mean_n = jnp.sum(onehot * mean[None, :], axis=1, keepdims=True)
    inv_n = jnp.sum(onehot * inv[None, :], axis=1, keepdims=True)
    out = (h3_ref[...] - mean_n) * inv_n * nw_ref[...] + nb_ref[...]
    out_ref[...] = jnp.maximum(out, 0.0)


def _row_spec(i_map=lambda i: (i, 0)):
    return pl.BlockSpec((_B, _D), i_map)


def _full(shape):
    return pl.BlockSpec(shape, lambda i: tuple(0 for _ in shape))


def kernel(node, edge_index, edge_attr, batch_ptr, eps,
           W1, b1, g1, be1, W2, b2, g2, be2, W3, b3, nw, nb):
    del edge_attr
    edges = edge_index.reshape(2, _NW, _NCHUNK, _CHUNK).transpose(1, 2, 0, 3)
    partials = _sc_agg()(node, edges)

    bp2 = batch_ptr.reshape(_N, 1)
    em1 = (eps - 1.0).reshape(1, 1)

    h3, stats = pl.pallas_call(
        _tc1_body,
        grid=(_NB,),
        in_specs=[
            _row_spec(),
            pl.BlockSpec((_NC, _B, _D), lambda i: (0, i, 0)),
            pl.BlockSpec((_B, 1), lambda i: (i, 0)),
            pl.BlockSpec(memory_space=pltpu.SMEM),
            _full((_D, _D)), _full((1, _D)), _full((1, _D)), _full((1, _D)),
            _full((_D, _D)), _full((1, _D)), _full((1, _D)), _full((1, _D)),
            _full((_D, _D)), _full((1, _D)),
        ],
        out_specs=[
            _row_spec(),
            _full((_G, _D)),
        ],
        out_shape=[
            jax.ShapeDtypeStruct((_N, _D), jnp.float32),
            jax.ShapeDtypeStruct((_G, _D), jnp.float32),
        ],
    )(node, partials, bp2, em1,
      W1, b1.reshape(1, _D), g1.reshape(1, _D), be1.reshape(1, _D),
      W2, b2.reshape(1, _D), g2.reshape(1, _D), be2.reshape(1, _D),
      W3, b3.reshape(1, _D))

    out = pl.pallas_call(
        _tc2_body,
        grid=(_NB,),
        in_specs=[
            _row_spec(),
            pl.BlockSpec((_B, 1), lambda i: (i, 0)),
            _full((_G, _D)),
            _full((1, _D)), _full((1, _D)),
        ],
        out_specs=_row_spec(),
        out_shape=jax.ShapeDtypeStruct((_N, _D), jnp.float32),
    )(h3, bp2, stats, nw.reshape(1, _D), nb.reshape(1, _D))
    return out


# speedup vs baseline: 12.8035x; 1.1492x over previous
import functools

import jax
import jax.numpy as jnp
from jax import lax
from jax.experimental import pallas as pl
from jax.experimental.pallas import tpu as pltpu
from jax.experimental.pallas import tpu_sc as plsc

_N = 10000
_E = 320000
_D = 128
_G = 16

_NC = 2
_NS = 16
_NW = _NC * _NS
_EPW = _E // _NW
_CHUNK = 80
_NCHUNK = _EPW // _CHUNK
_RPS = 624
_TAIL0 = _RPS * _NS
_TAILN = _N - _TAIL0

_B = 2000
_NB = _N // _B


def _sc_agg_body(node_hbm, edge_hbm, out_hbm,
                 ix0, ix1, ix2, ix3, buf0, buf1, buf2, buf3, agg_sh,
                 si0, si1, si2, si3, sg0, sg1, sg2, sg3):
    c = lax.axis_index("c")
    s = lax.axis_index("s")
    wid = c * _NS + s
    r0 = s * _RPS
    pltpu.sync_copy(node_hbm.at[pl.ds(r0, _RPS)], agg_sh.at[pl.ds(r0, _RPS)])

    @pl.when(s == 0)
    def _():
        pltpu.sync_copy(node_hbm.at[pl.ds(_TAIL0, _TAILN)],
                        agg_sh.at[pl.ds(_TAIL0, _TAILN)])

    plsc.subcore_barrier()

    ixs = (ix0, ix1, ix2, ix3)
    bufs = (buf0, buf1, buf2, buf3)
    sis = (si0, si1, si2, si3)
    sgs = (sg0, sg1, sg2, sg3)

    def fire_idx(m, k):
        pltpu.async_copy(edge_hbm.at[wid, m], ixs[k], sis[k])

    def wait_idx(k):
        pltpu.make_async_copy(edge_hbm.at[wid, 0], ixs[k], sis[k]).wait()

    def fire_gather(k):
        pltpu.async_copy(node_hbm.at[ixs[k].at[0]], bufs[k], sgs[k])

    def wait_gather(k):
        pltpu.make_async_copy(node_hbm.at[ixs[k].at[0]], bufs[k], sgs[k]).wait()

    for k in range(4):
        fire_idx(k, k)
    for k in range(2):
        wait_idx(k)
        fire_gather(k)

    def quad(i, carry):
        j = 4 * i
        for k in range(4):
            m = j + k

            @pl.when(m + 2 < _NCHUNK)
            def _():
                wait_idx((k + 2) % 4)
                fire_gather((k + 2) % 4)

            wait_gather(k)
            pltpu.sync_copy(bufs[k], agg_sh.at[ixs[k].at[1]], add=True)

            @pl.when(m + 4 < _NCHUNK)
            def _():
                fire_idx(m + 4, k)

        return carry

    lax.fori_loop(0, (_NCHUNK - 1) // 4, quad, 0)
    wait_gather(0)
    pltpu.sync_copy(bufs[0], agg_sh.at[ixs[0].at[1]], add=True)

    plsc.subcore_barrier()
    pltpu.sync_copy(agg_sh.at[pl.ds(r0, _RPS)], out_hbm.at[c, pl.ds(r0, _RPS)])

    @pl.when(s == 0)
    def _():
        pltpu.sync_copy(agg_sh.at[pl.ds(_TAIL0, _TAILN)],
                        out_hbm.at[c, pl.ds(_TAIL0, _TAILN)])


@functools.cache
def _sc_agg():
    return pl.kernel(
        _sc_agg_body,
        out_type=jax.ShapeDtypeStruct((_NC, _N, _D), jnp.float32),
        mesh=plsc.VectorSubcoreMesh(core_axis_name="c", subcore_axis_name="s"),
        scratch_types=(
            [pltpu.VMEM((2, _CHUNK), jnp.int32)] * 4
            + [pltpu.VMEM((_CHUNK, _D), jnp.float32)] * 4
            + [pltpu.VMEM_SHARED((_N, _D), jnp.float32)]
            + [pltpu.SemaphoreType.DMA] * 8
        ),
    )


def _ln(x, g, b):
    m = jnp.mean(x, axis=-1, keepdims=True)
    xc = x - m
    v = jnp.mean(xc * xc, axis=-1, keepdims=True)
    return xc * lax.rsqrt(v + 1e-5) * g + b


def _tc1_body(node_ref, p_ref, bp_ref, em1_ref,
              w1_ref, b1_ref, g1_ref, be1_ref,
              w2_ref, b2_ref, g2_ref, be2_ref,
              w3_ref, b3_ref, h3_ref, stats_ref):
    i = pl.program_id(0)
    h = em1_ref[0, 0] * node_ref[...] + p_ref[0] + p_ref[1]
    h = jnp.maximum(_ln(jnp.dot(h, w1_ref[...],
                                preferred_element_type=jnp.float32)
                        + b1_ref[...], g1_ref[...], be1_ref[...]), 0.0)
    h = jnp.maximum(_ln(jnp.dot(h, w2_ref[...],
                                preferred_element_type=jnp.float32)
                        + b2_ref[...], g2_ref[...], be2_ref[...]), 0.0)
    h = jnp.dot(h, w3_ref[...], preferred_element_type=jnp.float32) \
        + b3_ref[...]
    h3_ref[...] = h
    onehot = (bp_ref[...] == lax.broadcasted_iota(
        jnp.int32, (_B, _G), 1)).astype(jnp.float32)
    rs = jnp.sum(h, axis=1, keepdims=True)
    rs2 = jnp.sum(h * h, axis=1, keepdims=True)
    sum_g = jnp.sum(onehot * rs, axis=0)
    sum2_g = jnp.sum(onehot * rs2, axis=0)
    cnt_g = jnp.sum(onehot, axis=0)
    part = jnp.concatenate(
        [sum_g[:, None], sum2_g[:, None], cnt_g[:, None],
         jnp.zeros((_G, _D - 3), jnp.float32)], axis=1)

    @pl.when(i == 0)
    def _():
        stats_ref[...] = part

    @pl.when(i > 0)
    def _():
        stats_ref[...] += part


def _tc2_body(h3_ref, bp_ref, stats_ref, nw_ref, nb_ref, out_ref):
    stats = stats_ref[...]
    cnt = jnp.maximum(stats[:, 2] * jnp.float32(_D), 1.0)
    mean = stats[:, 0] / cnt
    var = jnp.maximum(stats[:, 1] / cnt - mean * mean, 0.0)
    inv = lax.rsqrt(var + 1e-5)
    onehot = (bp_ref[...] == lax.broadcasted_iota(
        jnp.int32, (_B, _G), 1)).astype(jnp.float32)
    mean_n = jnp.sum(onehot * mean[None, :], axis=1, keepdims=True)
    inv_n = jnp.sum(onehot * inv[None, :], axis=1, keepdims=True)
    out = (h3_ref[...] - mean_n) * inv_n * nw_ref[...] + nb_ref[...]
    out_ref[...] = jnp.maximum(out, 0.0)


def _row_spec(i_map=lambda i: (i, 0)):
    return pl.BlockSpec((_B, _D), i_map)


def _full(shape):
    return pl.BlockSpec(shape, lambda i: tuple(0 for _ in shape))


def kernel(node, edge_index, edge_attr, batch_ptr, eps,
           W1, b1, g1, be1, W2, b2, g2, be2, W3, b3, nw, nb):
    del edge_attr
    edges = edge_index.reshape(2, _NW, _NCHUNK, _CHUNK).transpose(1, 2, 0, 3)
    partials = _sc_agg()(node, edges)

    bp2 = batch_ptr.reshape(_N, 1)
    em1 = (eps - 1.0).reshape(1, 1)

    h3, stats = pl.pallas_call(
        _tc1_body,
        grid=(_NB,),
        in_specs=[
            _row_spec(),
            pl.BlockSpec((_NC, _B, _D), lambda i: (0, i, 0)),
            pl.BlockSpec((_B, 1), lambda i: (i, 0)),
            pl.BlockSpec(memory_space=pltpu.SMEM),
            _full((_D, _D)), _full((1, _D)), _full((1, _D)), _full((1, _D)),
            _full((_D, _D)), _full((1, _D)), _full((1, _D)), _full((1, _D)),
            _full((_D, _D)), _full((1, _D)),
        ],
        out_specs=[
            _row_spec(),
            _full((_G, _D)),
        ],
        out_shape=[
            jax.ShapeDtypeStruct((_N, _D), jnp.float32),
            jax.ShapeDtypeStruct((_G, _D), jnp.float32),
        ],
    )(node, partials, bp2, em1,
      W1, b1.reshape(1, _D), g1.reshape(1, _D), be1.reshape(1, _D),
      W2, b2.reshape(1, _D), g2.reshape(1, _D), be2.reshape(1, _D),
      W3, b3.reshape(1, _D))

    out = pl.pallas_call(
        _tc2_body,
        grid=(_NB,),
        in_specs=[
            _row_spec(),
            pl.BlockSpec((_B, 1), lambda i: (i, 0)),
            _full((_G, _D)),
            _full((1, _D)), _full((1, _D)),
        ],
        out_specs=_row_spec(),
        out_shape=jax.ShapeDtypeStruct((_N, _D), jnp.float32),
    )(h3, bp2, stats, nw.reshape(1, _D), nb.reshape(1, _D))
    return out


# speedup vs baseline: 12.8242x; 1.0016x over previous
import functools

import jax
import jax.numpy as jnp
from jax import lax
from jax.experimental import pallas as pl
from jax.experimental.pallas import tpu as pltpu
from jax.experimental.pallas import tpu_sc as plsc

_N = 10000
_E = 320000
_D = 128
_G = 16

_NC = 2
_NS = 16
_NW = _NC * _NS
_EPW = _E // _NW
_CHUNK = 80
_NCHUNK = _EPW // _CHUNK
_RPS = 624
_TAIL0 = _RPS * _NS
_TAILN = _N - _TAIL0

_B = 2000
_NB = _N // _B


def _sc_agg_body(node_hbm, edge_hbm, out_hbm,
                 ix0, ix1, ix2, ix3, buf0, buf1, buf2, buf3, agg_sh,
                 si0, si1, si2, si3, sg0, sg1, sg2, sg3):
    c = lax.axis_index("c")
    s = lax.axis_index("s")
    wid = c * _NS + s
    r0 = s * _RPS
    pltpu.sync_copy(node_hbm.at[pl.ds(r0, _RPS)], agg_sh.at[pl.ds(r0, _RPS)])

    @pl.when(s == 0)
    def _():
        pltpu.sync_copy(node_hbm.at[pl.ds(_TAIL0, _TAILN)],
                        agg_sh.at[pl.ds(_TAIL0, _TAILN)])

    plsc.subcore_barrier()

    ixs = (ix0, ix1, ix2, ix3)
    bufs = (buf0, buf1, buf2, buf3)
    sis = (si0, si1, si2, si3)
    sgs = (sg0, sg1, sg2, sg3)

    def fire_idx(m, k):
        pltpu.async_copy(edge_hbm.at[wid, m], ixs[k], sis[k])

    def wait_idx(k):
        pltpu.make_async_copy(edge_hbm.at[wid, 0], ixs[k], sis[k]).wait()

    def fire_gather(k):
        pltpu.async_copy(node_hbm.at[ixs[k].at[0]], bufs[k], sgs[k])

    def wait_gather(k):
        pltpu.make_async_copy(node_hbm.at[ixs[k].at[0]], bufs[k], sgs[k]).wait()

    for k in range(4):
        fire_idx(k, k)
    for k in range(2):
        wait_idx(k)
        fire_gather(k)

    def quad(i, carry):
        j = 4 * i
        for k in range(4):
            m = j + k

            @pl.when(m + 2 < _NCHUNK)
            def _():
                wait_idx((k + 2) % 4)
                fire_gather((k + 2) % 4)

            wait_gather(k)
            pltpu.sync_copy(bufs[k], agg_sh.at[ixs[k].at[1]], add=True)

            @pl.when(m + 4 < _NCHUNK)
            def _():
                fire_idx(m + 4, k)

        return carry

    lax.fori_loop(0, (_NCHUNK - 1) // 4, quad, 0)
    wait_gather(0)
    pltpu.sync_copy(bufs[0], agg_sh.at[ixs[0].at[1]], add=True)

    plsc.subcore_barrier()
    pltpu.sync_copy(agg_sh.at[pl.ds(r0, _RPS)], out_hbm.at[c, pl.ds(r0, _RPS)])

    @pl.when(s == 0)
    def _():
        pltpu.sync_copy(agg_sh.at[pl.ds(_TAIL0, _TAILN)],
                        out_hbm.at[c, pl.ds(_TAIL0, _TAILN)])


@functools.cache
def _sc_agg():
    return pl.kernel(
        _sc_agg_body,
        out_type=jax.ShapeDtypeStruct((_NC, _N, _D), jnp.float32),
        mesh=plsc.VectorSubcoreMesh(core_axis_name="c", subcore_axis_name="s"),
        scratch_types=(
            [pltpu.VMEM((2, _CHUNK), jnp.int32)] * 4
            + [pltpu.VMEM((_CHUNK, _D), jnp.float32)] * 4
            + [pltpu.VMEM_SHARED((_N, _D), jnp.float32)]
            + [pltpu.SemaphoreType.DMA] * 8
        ),
    )


def _ln(x, g, b):
    m = jnp.mean(x, axis=-1, keepdims=True)
    xc = x - m
    v = jnp.mean(xc * xc, axis=-1, keepdims=True)
    return xc * lax.rsqrt(v + 1e-5) * g + b


def _tc_body(node_ref, p_ref, bp_ref, em1_ref,
             w1_ref, b1_ref, g1_ref, be1_ref,
             w2_ref, b2_ref, g2_ref, be2_ref,
             w3_ref, b3_ref, nw_ref, nb_ref, out_ref,
             h3_scr, stats_scr):
    p = pl.program_id(0)
    i = pl.program_id(1)
    onehot = (bp_ref[...] == lax.broadcasted_iota(
        jnp.int32, (_B, _G), 1)).astype(jnp.float32)

    @pl.when(p == 0)
    def _():
        h = em1_ref[0, 0] * node_ref[...] + p_ref[0] + p_ref[1]
        h = jnp.maximum(_ln(jnp.dot(h, w1_ref[...],
                                    preferred_element_type=jnp.float32)
                            + b1_ref[...], g1_ref[...], be1_ref[...]), 0.0)
        h = jnp.maximum(_ln(jnp.dot(h, w2_ref[...],
                                    preferred_element_type=jnp.float32)
                            + b2_ref[...], g2_ref[...], be2_ref[...]), 0.0)
        h = jnp.dot(h, w3_ref[...], preferred_element_type=jnp.float32) \
            + b3_ref[...]
        h3_scr[pl.ds(i * _B, _B), :] = h
        rs = jnp.sum(h, axis=1, keepdims=True)
        rs2 = jnp.sum(h * h, axis=1, keepdims=True)
        part = jnp.concatenate(
            [jnp.sum(onehot * rs, axis=0)[:, None],
             jnp.sum(onehot * rs2, axis=0)[:, None],
             jnp.sum(onehot, axis=0)[:, None],
             jnp.zeros((_G, _D - 3), jnp.float32)], axis=1)

        @pl.when(i == 0)
        def _():
            stats_scr[...] = part

        @pl.when(i > 0)
        def _():
            stats_scr[...] += part

    @pl.when(p == 1)
    def _():
        stats = stats_scr[...]
        cnt = jnp.maximum(stats[:, 2] * jnp.float32(_D), 1.0)
        mean = stats[:, 0] / cnt
        var = jnp.maximum(stats[:, 1] / cnt - mean * mean, 0.0)
        inv = lax.rsqrt(var + 1e-5)
        mean_n = jnp.sum(onehot * mean[None, :], axis=1, keepdims=True)
        inv_n = jnp.sum(onehot * inv[None, :], axis=1, keepdims=True)
        h = h3_scr[pl.ds(i * _B, _B), :]
        out = (h - mean_n) * inv_n * nw_ref[...] + nb_ref[...]
        out_ref[...] = jnp.maximum(out, 0.0)


def _full(shape):
    return pl.BlockSpec(shape, lambda p, i: tuple(0 for _ in shape))


def kernel(node, edge_index, edge_attr, batch_ptr, eps,
           W1, b1, g1, be1, W2, b2, g2, be2, W3, b3, nw, nb):
    del edge_attr
    edges = edge_index.reshape(2, _NW, _NCHUNK, _CHUNK).transpose(1, 2, 0, 3)
    partials = _sc_agg()(node, edges)

    bp2 = batch_ptr.reshape(_N, 1)
    em1 = (eps - 1.0).reshape(1, 1)

    out = pl.pallas_call(
        _tc_body,
        grid=(2, _NB),
        in_specs=[
            pl.BlockSpec((_B, _D), lambda p, i: (i, 0)),
            pl.BlockSpec((_NC, _B, _D), lambda p, i: (0, i, 0)),
            pl.BlockSpec((_B, 1), lambda p, i: (i, 0)),
            pl.BlockSpec(memory_space=pltpu.SMEM),
            _full((_D, _D)), _full((1, _D)), _full((1, _D)), _full((1, _D)),
            _full((_D, _D)), _full((1, _D)), _full((1, _D)), _full((1, _D)),
            _full((_D, _D)), _full((1, _D)), _full((1, _D)), _full((1, _D)),
        ],
        out_specs=pl.BlockSpec((_B, _D), lambda p, i: (i, 0)),
        out_shape=jax.ShapeDtypeStruct((_N, _D), jnp.float32),
        scratch_shapes=[
            pltpu.VMEM((_N, _D), jnp.float32),
            pltpu.VMEM((_G, _D), jnp.float32),
        ],
    )(node, partials, bp2, em1,
      W1, b1.reshape(1, _D), g1.reshape(1, _D), be1.reshape(1, _D),
      W2, b2.reshape(1, _D), g2.reshape(1, _D), be2.reshape(1, _D),
      W3, b3.reshape(1, _D), nw.reshape(1, _D), nb.reshape(1, _D))
    return out


# speedup vs baseline: 13.3833x; 1.0436x over previous
import functools

import jax
import jax.numpy as jnp
from jax import lax
from jax.experimental import pallas as pl
from jax.experimental.pallas import tpu as pltpu
from jax.experimental.pallas import tpu_sc as plsc

_N = 10000
_E = 320000
_D = 128
_G = 16

_NC = 2
_NS = 16
_NW = _NC * _NS
_EPW = _E // _NW
_CHUNK = 80
_NCHUNK = _EPW // _CHUNK
_RPS = 624
_TAIL0 = _RPS * _NS
_TAILN = _N - _TAIL0

_B = 2000
_NB = _N // _B


def _sc_agg_body(node_hbm, edge_hbm, out_hbm,
                 ix0, ix1, ix2, ix3, buf0, buf1, buf2, buf3, agg_sh,
                 si0, si1, si2, si3, sg0, sg1, sg2, sg3,
                 ss0, ss1, ss2, ss3, sm):
    c = lax.axis_index("c")
    s = lax.axis_index("s")
    wid = c * _NS + s
    r0 = s * _RPS

    ixs = (ix0, ix1, ix2, ix3)
    bufs = (buf0, buf1, buf2, buf3)
    sis = (si0, si1, si2, si3)
    sgs = (sg0, sg1, sg2, sg3)
    sss = (ss0, ss1, ss2, ss3)

    def fire_idx(m, k):
        off = wid * _EPW + m * _CHUNK
        pltpu.async_copy(edge_hbm.at[pl.ds(off, _CHUNK)],
                         ixs[k].at[0], sis[k])
        pltpu.async_copy(edge_hbm.at[pl.ds(_E + off, _CHUNK)],
                         ixs[k].at[1], sis[k])

    def wait_idx(k):
        pltpu.make_async_copy(edge_hbm.at[pl.ds(0, _CHUNK)],
                              ixs[k].at[0], sis[k]).wait()
        pltpu.make_async_copy(edge_hbm.at[pl.ds(0, _CHUNK)],
                              ixs[k].at[1], sis[k]).wait()

    def fire_gather(k):
        pltpu.async_copy(node_hbm.at[ixs[k].at[0]], bufs[k], sgs[k])

    def wait_gather(k):
        pltpu.make_async_copy(node_hbm.at[ixs[k].at[0]], bufs[k], sgs[k]).wait()

    pltpu.async_copy(node_hbm.at[pl.ds(r0, _RPS)],
                     agg_sh.at[pl.ds(r0, _RPS)], sm).wait()

    @pl.when(s == 0)
    def _():
        pltpu.async_copy(node_hbm.at[pl.ds(_TAIL0, _TAILN)],
                         agg_sh.at[pl.ds(_TAIL0, _TAILN)], sm).wait()

    plsc.subcore_barrier()

    for k in range(4):
        fire_idx(k, k)
    for k in range(2):
        wait_idx(k)
        fire_gather(k)

    def quad(i, carry):
        j = 4 * i
        for k in range(4):
            m = j + k

            @pl.when(m + 2 < _NCHUNK)
            def _():
                wait_idx((k + 2) % 4)
                fire_gather((k + 2) % 4)

            wait_gather(k)
            pltpu.async_copy(bufs[k], agg_sh.at[ixs[k].at[1]], sss[k],
                             add=True).wait()

            @pl.when(m + 4 < _NCHUNK)
            def _():
                fire_idx(m + 4, k)

        return carry

    lax.fori_loop(0, (_NCHUNK - 1) // 4, quad, 0)
    wait_gather(0)
    pltpu.async_copy(bufs[0], agg_sh.at[ixs[0].at[1]], sss[0], add=True).wait()

    plsc.subcore_barrier()
    pltpu.async_copy(agg_sh.at[pl.ds(r0, _RPS)],
                     out_hbm.at[c, pl.ds(r0, _RPS)], sm).wait()

    @pl.when(s == 0)
    def _():
        pltpu.async_copy(agg_sh.at[pl.ds(_TAIL0, _TAILN)],
                         out_hbm.at[c, pl.ds(_TAIL0, _TAILN)], sm).wait()


@functools.cache
def _sc_agg():
    return pl.kernel(
        _sc_agg_body,
        out_type=jax.ShapeDtypeStruct((_NC, _N, _D), jnp.float32),
        mesh=plsc.VectorSubcoreMesh(core_axis_name="c", subcore_axis_name="s"),
        scratch_types=(
            [pltpu.VMEM((2, _CHUNK), jnp.int32)] * 4
            + [pltpu.VMEM((_CHUNK, _D), jnp.float32)] * 4
            + [pltpu.VMEM_SHARED((_N, _D), jnp.float32)]
            + [pltpu.SemaphoreType.DMA] * 13
        ),
    )


def _ln(x, g, b):
    m = jnp.mean(x, axis=-1, keepdims=True)
    xc = x - m
    v = jnp.mean(xc * xc, axis=-1, keepdims=True)
    return xc * lax.rsqrt(v + 1e-5) * g + b


def _tc_body(node_ref, p_ref, bp_ref, em1_ref,
             w1_ref, b1_ref, g1_ref, be1_ref,
             w2_ref, b2_ref, g2_ref, be2_ref,
             w3_ref, b3_ref, nw_ref, nb_ref, out_ref,
             h3_scr, stats_scr):
    p = pl.program_id(0)
    i = pl.program_id(1)
    onehot = (bp_ref[...] == lax.broadcasted_iota(
        jnp.int32, (_B, _G), 1)).astype(jnp.float32)

    @pl.when(p == 0)
    def _():
        h = em1_ref[0, 0] * node_ref[...] + p_ref[0] + p_ref[1]
        h = jnp.maximum(_ln(jnp.dot(h, w1_ref[...],
                                    preferred_element_type=jnp.float32)
                            + b1_ref[...], g1_ref[...], be1_ref[...]), 0.0)
        h = jnp.maximum(_ln(jnp.dot(h, w2_ref[...],
                                    preferred_element_type=jnp.float32)
                            + b2_ref[...], g2_ref[...], be2_ref[...]), 0.0)
        h = jnp.dot(h, w3_ref[...], preferred_element_type=jnp.float32) \
            + b3_ref[...]
        h3_scr[pl.ds(i * _B, _B), :] = h
        rs = jnp.sum(h, axis=1, keepdims=True)
        rs2 = jnp.sum(h * h, axis=1, keepdims=True)
        part = jnp.concatenate(
            [jnp.sum(onehot * rs, axis=0)[:, None],
             jnp.sum(onehot * rs2, axis=0)[:, None],
             jnp.sum(onehot, axis=0)[:, None],
             jnp.zeros((_G, _D - 3), jnp.float32)], axis=1)

        @pl.when(i == 0)
        def _():
            stats_scr[...] = part

        @pl.when(i > 0)
        def _():
            stats_scr[...] += part

    @pl.when(p == 1)
    def _():
        stats = stats_scr[...]
        cnt = jnp.maximum(stats[:, 2] * jnp.float32(_D), 1.0)
        mean = stats[:, 0] / cnt
        var = jnp.maximum(stats[:, 1] / cnt - mean * mean, 0.0)
        inv = lax.rsqrt(var + 1e-5)
        mean_n = jnp.sum(onehot * mean[None, :], axis=1, keepdims=True)
        inv_n = jnp.sum(onehot * inv[None, :], axis=1, keepdims=True)
        h = h3_scr[pl.ds(i * _B, _B), :]
        out = (h - mean_n) * inv_n * nw_ref[...] + nb_ref[...]
        out_ref[...] = jnp.maximum(out, 0.0)


def _full(shape):
    return pl.BlockSpec(shape, lambda p, i: tuple(0 for _ in shape))


def kernel(node, edge_index, edge_attr, batch_ptr, eps,
           W1, b1, g1, be1, W2, b2, g2, be2, W3, b3, nw, nb):
    del edge_attr
    partials = _sc_agg()(node, edge_index.reshape(2 * _E))

    bp2 = batch_ptr.reshape(_N, 1)
    em1 = (eps - 1.0).reshape(1, 1)

    out = pl.pallas_call(
        _tc_body,
        grid=(2, _NB),
        in_specs=[
            pl.BlockSpec((_B, _D), lambda p, i: (i, 0)),
            pl.BlockSpec((_NC, _B, _D), lambda p, i: (0, i, 0)),
            pl.BlockSpec((_B, 1), lambda p, i: (i, 0)),
            pl.BlockSpec(memory_space=pltpu.SMEM),
            _full((_D, _D)), _full((1, _D)), _full((1, _D)), _full((1, _D)),
            _full((_D, _D)), _full((1, _D)), _full((1, _D)), _full((1, _D)),
            _full((_D, _D)), _full((1, _D)), _full((1, _D)), _full((1, _D)),
        ],
        out_specs=pl.BlockSpec((_B, _D), lambda p, i: (i, 0)),
        out_shape=jax.ShapeDtypeStruct((_N, _D), jnp.float32),
        scratch_shapes=[
            pltpu.VMEM((_N, _D), jnp.float32),
            pltpu.VMEM((_G, _D), jnp.float32),
        ],
    )(node, partials, bp2, em1,
      W1, b1.reshape(1, _D), g1.reshape(1, _D), be1.reshape(1, _D),
      W2, b2.reshape(1, _D), g2.reshape(1, _D), be2.reshape(1, _D),
      W3, b3.reshape(1, _D), nw.reshape(1, _D), nb.reshape(1, _D))
    return out
